# baseline (speedup 1.0000x reference)
import functools

import jax
import jax.numpy as jnp
from jax import lax
from jax.experimental import pallas as pl
from jax.experimental.pallas import tpu as pltpu
from jax.experimental.pallas import tpu_sc as plsc

M = 100000
B = 16384
D = 128
C = 100
CP = 128

NS = 16
CH = B // NS
NV = CH // 16
W = 100352
DUM = 100096
ROUNDS = 8
HALF = M // 2
ESZ = 3128
ESZ_L = HALF - 15 * ESZ
WIN = 136
NWIN = ESZ // WIN
WIN_L = ESZ_L - 22 * WIN
NCHK = 8


def _body(bx_h, lb_h, by_h, bt_h, x_h, ln_h, byn_h, idx_h, t_h,
          obx_h, oby_h, obt_h, olg_h, w_h,
          idx_v, ival_v, sidx_v, rv_v, s2_v, fill_v,
          cdst1, csrc1, cdst2, csrc2, byv, tv, t_v,
          bxw, lgw, xrow, lrow, byc, sem):
    c = lax.axis_index("c")
    s = lax.axis_index("s")
    g = c * NS + s

    pltpu.sync_copy(idx_h.at[pl.ds(s * CH, CH)], idx_v)
    pltpu.sync_copy(t_h, t_v)
    wbase = c * W

    def f0(j, _):
        sl = pl.ds(j * 16, 16)
        iv = lax.iota(jnp.int32, 16) + (s * CH + j * 16)
        ival_v[sl] = iv
        ix = idx_v[sl]
        dum = wbase + DUM + (iv & 255)
        sidx_v[sl] = jnp.where(ix < M, wbase + ix, dum)
        tv[sl] = t_v[...]
        return 0

    lax.fori_loop(0, NV, f0, 0)

    neg = jnp.full((16,), -1, jnp.int32)

    def f1(j, _):
        fill_v[pl.ds(j * 16, 16)] = neg
        return 0

    lax.fori_loop(0, W // NS // 16, f1, 0)
    pltpu.sync_copy(fill_v, w_h.at[pl.ds(wbase + s * (W // NS), W // NS)])
    plsc.subcore_barrier()

    pltpu.async_copy(ival_v, w_h.at[sidx_v], sem).wait()
    plsc.subcore_barrier()
    for _ in range(ROUNDS):
        pltpu.async_copy(w_h.at[sidx_v], rv_v, sem).wait()
        plsc.subcore_barrier()

        def f2(j, _):
            sl = pl.ds(j * 16, 16)
            ix = idx_v[sl]
            iv = ival_v[sl]
            need = (ix < M) & (rv_v[sl] < iv)
            dum = wbase + DUM + (iv & 255)
            s2_v[sl] = jnp.where(need, sidx_v[sl], dum)
            return 0

        lax.fori_loop(0, NV, f2, 0)
        pltpu.async_copy(ival_v, w_h.at[s2_v], sem).wait()
        plsc.subcore_barrier()

    pltpu.async_copy(w_h.at[sidx_v], rv_v, sem).wait()
    lo = c * HALF

    def f3(j, cnt):
        sl = pl.ds(j * 16, 16)
        ix = idx_v[sl]
        iv = ival_v[sl]
        win = (ix < M) & (rv_v[sl] == iv) & (ix >= lo) & (ix < lo + HALF)
        wm = jnp.where(win, 1, 0)
        inc = plsc.cumsum(wm)
        pos = cnt + (inc - wm)
        plsc.store_scatter(cdst1, [pos], ix, mask=win)
        plsc.store_scatter(csrc1, [pos], iv, mask=win)
        plsc.store_scatter(cdst2, [pos >> 7, pos & 127], ix, mask=win)
        plsc.store_scatter(csrc2, [pos >> 7, pos & 127], iv, mask=win)
        return cnt + jnp.sum(wm)

    cnt = lax.fori_loop(0, NV, f3, 0)

    @pl.when(cnt > 0)
    def _():
        d0 = cdst1[pl.ds(0, 16)][0]
        s0 = csrc1[pl.ds(0, 16)][0]

        def f4(j, _):
            sl = pl.ds(j * 16, 16)
            posv = lax.iota(jnp.int32, 16) + j * 16
            sel = posv < cnt
            cd = jnp.where(sel, cdst1[sl], d0)
            cs = jnp.where(sel, csrc1[sl], s0)
            cdst1[sl] = cd
            csrc1[sl] = cs
            plsc.store_scatter(cdst2, [posv >> 7, posv & 127], cd)
            plsc.store_scatter(csrc2, [posv >> 7, posv & 127], cs)
            return 0

        lax.fori_loop(0, NV, f4, 0)

    e0 = c * HALF + s * ESZ

    def f5(wi, _):
        r = e0 + wi * WIN
        pltpu.sync_copy(bx_h.at[pl.ds(r, WIN)], bxw)
        pltpu.sync_copy(bxw, obx_h.at[pl.ds(r, WIN)])
        pltpu.sync_copy(lb_h.at[pl.ds(r, WIN)], lgw)
        pltpu.sync_copy(lgw, olg_h.at[pl.ds(r, WIN)])
        return 0

    nwin = jnp.where(s < NS - 1, NWIN, NWIN - 1)
    lax.fori_loop(0, nwin, f5, 0)

    @pl.when(s == NS - 1)
    def _():
        r = e0 + (NWIN - 1) * WIN
        pltpu.sync_copy(bx_h.at[pl.ds(r, WIN_L)], bxw.at[pl.ds(0, WIN_L)])
        pltpu.sync_copy(bxw.at[pl.ds(0, WIN_L)], obx_h.at[pl.ds(r, WIN_L)])
        pltpu.sync_copy(lb_h.at[pl.ds(r, WIN_L)], lgw.at[pl.ds(0, WIN_L)])
        pltpu.sync_copy(lgw.at[pl.ds(0, WIN_L)], olg_h.at[pl.ds(r, WIN_L)])

    @pl.when(s < NS - 1)
    def _():
        pltpu.sync_copy(by_h.at[pl.ds(e0, ESZ)], byc)
        pltpu.sync_copy(byc, oby_h.at[pl.ds(e0, ESZ)])
        pltpu.sync_copy(bt_h.at[pl.ds(e0, ESZ)], byc)
        pltpu.sync_copy(byc, obt_h.at[pl.ds(e0, ESZ)])

    @pl.when(s == NS - 1)
    def _():
        pltpu.sync_copy(by_h.at[pl.ds(e0, ESZ_L)], byc.at[pl.ds(0, ESZ_L)])
        pltpu.sync_copy(byc.at[pl.ds(0, ESZ_L)], oby_h.at[pl.ds(e0, ESZ_L)])
        pltpu.sync_copy(bt_h.at[pl.ds(e0, ESZ_L)], byc.at[pl.ds(0, ESZ_L)])
        pltpu.sync_copy(byc.at[pl.ds(0, ESZ_L)], obt_h.at[pl.ds(e0, ESZ_L)])

    plsc.subcore_barrier()

    @pl.when(cnt > 0)
    def _():
        pltpu.async_copy(byn_h.at[csrc1], byv, sem).wait()
        pltpu.async_copy(byv, oby_h.at[cdst1], sem).wait()
        pltpu.async_copy(tv, obt_h.at[cdst1], sem).wait()

    for k in range(NCHK):
        @pl.when(cnt > k * 128)
        def _(k=k):
            pltpu.async_copy(x_h.at[csrc2.at[k]], xrow, sem).wait()
            pltpu.async_copy(xrow, obx_h.at[cdst2.at[k]], sem).wait()
            pltpu.async_copy(ln_h.at[csrc2.at[k]], lrow, sem).wait()
            pltpu.async_copy(lrow, olg_h.at[cdst2.at[k]], sem).wait()


@functools.partial(jax.jit, static_argnames=())
def _run(bx, logits_buf, by_buf, bt_buf, x, logits_new, by_new, idx, tarr):
    f = functools.partial(
        pl.kernel,
        mesh=plsc.VectorSubcoreMesh(core_axis_name="c", subcore_axis_name="s"),
        compiler_params=pltpu.CompilerParams(needs_layout_passes=False),
        out_type=[
            jax.ShapeDtypeStruct((M, D), jnp.float32),
            jax.ShapeDtypeStruct((M,), jnp.int32),
            jax.ShapeDtypeStruct((M,), jnp.int32),
            jax.ShapeDtypeStruct((M, CP), jnp.float32),
            jax.ShapeDtypeStruct((2 * W,), jnp.int32),
        ],
        scratch_types=[
            pltpu.VMEM((CH,), jnp.int32),
            pltpu.VMEM((CH,), jnp.int32),
            pltpu.VMEM((CH,), jnp.int32),
            pltpu.VMEM((CH,), jnp.int32),
            pltpu.VMEM((CH,), jnp.int32),
            pltpu.VMEM((W // NS,), jnp.int32),
            pltpu.VMEM((CH,), jnp.int32),
            pltpu.VMEM((CH,), jnp.int32),
            pltpu.VMEM((NCHK, 128), jnp.int32),
            pltpu.VMEM((NCHK, 128), jnp.int32),
            pltpu.VMEM((CH,), jnp.int32),
            pltpu.VMEM((CH,), jnp.int32),
            pltpu.VMEM((16,), jnp.int32),
            pltpu.VMEM((WIN, D), jnp.float32),
            pltpu.VMEM((WIN, CP), jnp.float32),
            pltpu.VMEM((128, D), jnp.float32),
            pltpu.VMEM((128, CP), jnp.float32),
            pltpu.VMEM((ESZ,), jnp.int32),
            pltpu.SemaphoreType.DMA,
        ],
    )(_body)
    obx, oby, obt, olgp, _w = f(bx, logits_buf, by_buf, bt_buf,
                                x, logits_new, by_new, idx, tarr)
    return obx, oby, obt, olgp[:, :C]


def kernel(bx, logits_buf, by_buf, bt_buf, x, logits_new, by_new, idx, t):
    tarr = jnp.full((16,), t, jnp.int32)
    lbp = jnp.pad(logits_buf, ((0, 0), (0, CP - C)))
    lnp = jnp.pad(logits_new, ((0, 0), (0, CP - C)))
    return _run(bx, lbp, by_buf, bt_buf, x, lnp, by_new, idx, tarr)


# speedup vs baseline: 7.4119x; 7.4119x over previous
import functools

import jax
import jax.numpy as jnp
from jax import lax
from jax.experimental import pallas as pl
from jax.experimental.pallas import tpu as pltpu
from jax.experimental.pallas import tpu_sc as plsc

M = 100000
B = 16384
D = 128
C = 100
CP = 128

NS = 16
CH = B // NS
NV = CH // 16
W = 108544
DUM = 100096
ROUNDS = 5
HALF = M // 2
ESZ = 3128
ESZ_L = HALF - 15 * ESZ
WIN = 136
NWIN = ESZ // WIN
WIN_L = ESZ_L - 22 * WIN
NCHK = 8


def _body(bx_h, lb_h, by_h, bt_h, x_h, ln_h, byn_h, idx_h, t_h,
          obx_h, oby_h, obt_h, olg_h, w_h,
          idx_v, ival_v, sidx_v, rv_v, s2_v, fill_v,
          cdst1, csrc1, cdst2, csrc2, byv, tv, t_v,
          bxw, lgw, xrow, lrow, byc, sem):
    c = lax.axis_index("c")
    s = lax.axis_index("s")
    g = c * NS + s

    pltpu.sync_copy(idx_h.at[pl.ds(s * CH, CH)], idx_v)
    pltpu.sync_copy(t_h, t_v)
    wbase = c * W

    def f0(j, _):
        sl = pl.ds(j * 16, 16)
        iv = lax.iota(jnp.int32, 16) + (s * CH + j * 16)
        ival_v[sl] = iv
        ix = idx_v[sl]
        dum = wbase + DUM + s * 512 + (iv & 511)
        sidx_v[sl] = jnp.where(ix < M, wbase + ix, dum)
        tv[sl] = t_v[...]
        return 0

    lax.fori_loop(0, NV, f0, 0)

    neg = jnp.full((16,), -1, jnp.int32)

    def f1(j, _):
        fill_v[pl.ds(j * 16, 16)] = neg
        return 0

    lax.fori_loop(0, W // NS // 16, f1, 0)
    pltpu.sync_copy(fill_v, w_h.at[pl.ds(wbase + s * (W // NS), W // NS)])
    plsc.subcore_barrier()

    pltpu.async_copy(ival_v, w_h.at[sidx_v], sem).wait()
    plsc.subcore_barrier()
    for _ in range(ROUNDS):
        pltpu.async_copy(w_h.at[sidx_v], rv_v, sem).wait()
        plsc.subcore_barrier()

        def f2(j, _):
            sl = pl.ds(j * 16, 16)
            ix = idx_v[sl]
            iv = ival_v[sl]
            need = (ix < M) & (rv_v[sl] < iv)
            dum = wbase + DUM + s * 512 + (iv & 511)
            s2_v[sl] = jnp.where(need, sidx_v[sl], dum)
            return 0

        lax.fori_loop(0, NV, f2, 0)
        pltpu.async_copy(ival_v, w_h.at[s2_v], sem).wait()
        plsc.subcore_barrier()

    pltpu.async_copy(w_h.at[sidx_v], rv_v, sem).wait()
    lo = c * HALF

    def f3(j, cnt):
        sl = pl.ds(j * 16, 16)
        ix = idx_v[sl]
        iv = ival_v[sl]
        win = (ix < M) & (rv_v[sl] == iv) & (ix >= lo) & (ix < lo + HALF)
        wm = jnp.where(win, 1, 0)
        inc = plsc.cumsum(wm)
        pos = cnt + (inc - wm)
        plsc.store_scatter(cdst1, [pos], ix, mask=win)
        plsc.store_scatter(csrc1, [pos], iv, mask=win)
        plsc.store_scatter(cdst2, [pos >> 7, pos & 127], ix, mask=win)
        plsc.store_scatter(csrc2, [pos >> 7, pos & 127], iv, mask=win)
        return cnt + jnp.sum(wm)

    cnt = lax.fori_loop(0, NV, f3, 0)

    @pl.when(cnt > 0)
    def _():
        d0 = cdst1[pl.ds(0, 16)][0]
        s0 = csrc1[pl.ds(0, 16)][0]

        def f4(j, _):
            sl = pl.ds(j * 16, 16)
            posv = lax.iota(jnp.int32, 16) + j * 16
            sel = posv < cnt
            cd = jnp.where(sel, cdst1[sl], d0)
            cs = jnp.where(sel, csrc1[sl], s0)
            cdst1[sl] = cd
            csrc1[sl] = cs
            plsc.store_scatter(cdst2, [posv >> 7, posv & 127], cd)
            plsc.store_scatter(csrc2, [posv >> 7, posv & 127], cs)
            return 0

        lax.fori_loop(0, NV, f4, 0)

    e0 = c * HALF + s * ESZ

    def f5(wi, _):
        r = e0 + wi * WIN
        pltpu.sync_copy(bx_h.at[pl.ds(r, WIN)], bxw)
        pltpu.sync_copy(bxw, obx_h.at[pl.ds(r, WIN)])
        pltpu.sync_copy(lb_h.at[pl.ds(r, WIN)], lgw)
        pltpu.sync_copy(lgw, olg_h.at[pl.ds(r, WIN)])
        return 0

    nwin = jnp.where(s < NS - 1, NWIN, NWIN - 1)
    lax.fori_loop(0, nwin, f5, 0)

    @pl.when(s == NS - 1)
    def _():
        r = e0 + (NWIN - 1) * WIN
        pltpu.sync_copy(bx_h.at[pl.ds(r, WIN_L)], bxw.at[pl.ds(0, WIN_L)])
        pltpu.sync_copy(bxw.at[pl.ds(0, WIN_L)], obx_h.at[pl.ds(r, WIN_L)])
        pltpu.sync_copy(lb_h.at[pl.ds(r, WIN_L)], lgw.at[pl.ds(0, WIN_L)])
        pltpu.sync_copy(lgw.at[pl.ds(0, WIN_L)], olg_h.at[pl.ds(r, WIN_L)])

    @pl.when(s < NS - 1)
    def _():
        pltpu.sync_copy(by_h.at[pl.ds(e0, ESZ)], byc)
        pltpu.sync_copy(byc, oby_h.at[pl.ds(e0, ESZ)])
        pltpu.sync_copy(bt_h.at[pl.ds(e0, ESZ)], byc)
        pltpu.sync_copy(byc, obt_h.at[pl.ds(e0, ESZ)])

    @pl.when(s == NS - 1)
    def _():
        pltpu.sync_copy(by_h.at[pl.ds(e0, ESZ_L)], byc.at[pl.ds(0, ESZ_L)])
        pltpu.sync_copy(byc.at[pl.ds(0, ESZ_L)], oby_h.at[pl.ds(e0, ESZ_L)])
        pltpu.sync_copy(bt_h.at[pl.ds(e0, ESZ_L)], byc.at[pl.ds(0, ESZ_L)])
        pltpu.sync_copy(byc.at[pl.ds(0, ESZ_L)], obt_h.at[pl.ds(e0, ESZ_L)])

    plsc.subcore_barrier()

    @pl.when(cnt > 0)
    def _():
        pltpu.async_copy(byn_h.at[csrc1], byv, sem).wait()
        pltpu.async_copy(byv, oby_h.at[cdst1], sem).wait()
        pltpu.async_copy(tv, obt_h.at[cdst1], sem).wait()

    for k in range(NCHK):
        @pl.when(cnt > k * 128)
        def _(k=k):
            pltpu.async_copy(x_h.at[csrc2.at[k]], xrow, sem).wait()
            pltpu.async_copy(xrow, obx_h.at[cdst2.at[k]], sem).wait()
            pltpu.async_copy(ln_h.at[csrc2.at[k]], lrow, sem).wait()
            pltpu.async_copy(lrow, olg_h.at[cdst2.at[k]], sem).wait()


@functools.partial(jax.jit, static_argnames=())
def _run(bx, logits_buf, by_buf, bt_buf, x, logits_new, by_new, idx, tarr):
    f = functools.partial(
        pl.kernel,
        mesh=plsc.VectorSubcoreMesh(core_axis_name="c", subcore_axis_name="s"),
        compiler_params=pltpu.CompilerParams(needs_layout_passes=False),
        out_type=[
            jax.ShapeDtypeStruct((M, D), jnp.float32),
            jax.ShapeDtypeStruct((M,), jnp.int32),
            jax.ShapeDtypeStruct((M,), jnp.int32),
            jax.ShapeDtypeStruct((M, CP), jnp.float32),
            jax.ShapeDtypeStruct((2 * W,), jnp.int32),
        ],
        scratch_types=[
            pltpu.VMEM((CH,), jnp.int32),
            pltpu.VMEM((CH,), jnp.int32),
            pltpu.VMEM((CH,), jnp.int32),
            pltpu.VMEM((CH,), jnp.int32),
            pltpu.VMEM((CH,), jnp.int32),
            pltpu.VMEM((W // NS,), jnp.int32),
            pltpu.VMEM((CH,), jnp.int32),
            pltpu.VMEM((CH,), jnp.int32),
            pltpu.VMEM((NCHK, 128), jnp.int32),
            pltpu.VMEM((NCHK, 128), jnp.int32),
            pltpu.VMEM((CH,), jnp.int32),
            pltpu.VMEM((CH,), jnp.int32),
            pltpu.VMEM((16,), jnp.int32),
            pltpu.VMEM((WIN, D), jnp.float32),
            pltpu.VMEM((WIN, CP), jnp.float32),
            pltpu.VMEM((128, D), jnp.float32),
            pltpu.VMEM((128, CP), jnp.float32),
            pltpu.VMEM((ESZ,), jnp.int32),
            pltpu.SemaphoreType.DMA,
        ],
    )(_body)
    obx, oby, obt, olgp, _w = f(bx, logits_buf, by_buf, bt_buf,
                                x, logits_new, by_new, idx, tarr)
    return obx, oby, obt, olgp[:, :C]


def kernel(bx, logits_buf, by_buf, bt_buf, x, logits_new, by_new, idx, t):
    tarr = jnp.full((16,), t, jnp.int32)
    lbp = jnp.pad(logits_buf, ((0, 0), (0, CP - C)))
    lnp = jnp.pad(logits_new, ((0, 0), (0, CP - C)))
    return _run(bx, lbp, by_buf, bt_buf, x, lnp, by_new, idx, tarr)
